# speedup vs baseline: 70.8444x; 70.8444x over previous
import dataclasses
import functools

import jax
import jax.numpy as jnp
from jax import lax
from jax.experimental import pallas as pl
from jax.experimental.pallas import tpu as pltpu
from jax.experimental.pallas import tpu_sc as plsc

NUM_WORKERS = 32
_LANES = 16

_cp = pltpu.CompilerParams()
if "needs_layout_passes" in pltpu.CompilerParams.__dataclass_fields__:
    _cp = dataclasses.replace(_cp, needs_layout_passes=False)

_MESH = plsc.VectorSubcoreMesh(core_axis_name="c", subcore_axis_name="s")

_HIGH = jax.lax.Precision.HIGHEST


@functools.lru_cache(maxsize=None)
def _make_deg_kernel(n_nodes: int, n_edges: int):
    ep = n_edges // NUM_WORKERS
    assert n_edges % NUM_WORKERS == 0 and ep % _LANES == 0
    assert n_nodes % (5 * _LANES) == 0

    @functools.partial(
        pl.kernel,
        out_type=jax.ShapeDtypeStruct((NUM_WORKERS, n_nodes), jnp.float32),
        mesh=_MESH,
        scratch_types=[
            pltpu.VMEM((ep,), jnp.int32),
            pltpu.VMEM((n_nodes,), jnp.float32),
        ],
        compiler_params=_cp,
    )
    def deg_kernel(dst_hbm, out_hbm, dst_v, deg_v):
        wid = lax.axis_index("s") * 2 + lax.axis_index("c")
        pltpu.sync_copy(dst_hbm.at[pl.ds(wid * ep, ep)], dst_v)

        zero16 = jnp.zeros((_LANES,), jnp.float32)

        @pl.loop(0, n_nodes, step=5 * _LANES)
        def _(i):
            for u in range(5):
                deg_v[pl.ds(i + u * _LANES, _LANES)] = zero16

        ones16 = jnp.ones((_LANES,), jnp.float32)

        @pl.loop(0, ep, step=_LANES)
        def _(i):
            d = dst_v[pl.ds(i, _LANES)]
            plsc.addupdate_scatter(deg_v, [d], ones16)

        pltpu.sync_copy(deg_v, out_hbm.at[wid])

    return deg_kernel


@functools.lru_cache(maxsize=None)
def _make_agg_kernel(n_nodes: int, n_edges: int, w: int):
    ep = n_edges // NUM_WORKERS
    tbl = w * n_nodes
    assert n_edges % NUM_WORKERS == 0 and ep % _LANES == 0
    assert tbl % (5 * _LANES) == 0

    @functools.partial(
        pl.kernel,
        out_type=jax.ShapeDtypeStruct((NUM_WORKERS, tbl), jnp.float32),
        mesh=_MESH,
        scratch_types=[
            pltpu.VMEM((ep,), jnp.int32),
            pltpu.VMEM((ep,), jnp.int32),
            pltpu.VMEM((tbl,), jnp.float32),
            pltpu.VMEM((tbl,), jnp.float32),
        ],
        compiler_params=_cp,
    )
    def agg_kernel(z_hbm, src_hbm, dst_hbm, out_hbm, src_v, dst_v, z_v, acc_v):
        wid = lax.axis_index("s") * 2 + lax.axis_index("c")
        base = wid * ep
        pltpu.sync_copy(src_hbm.at[pl.ds(base, ep)], src_v)
        pltpu.sync_copy(dst_hbm.at[pl.ds(base, ep)], dst_v)
        pltpu.sync_copy(z_hbm, z_v)

        zero16 = jnp.zeros((_LANES,), jnp.float32)

        @pl.loop(0, tbl, step=5 * _LANES)
        def _(i):
            for u in range(5):
                acc_v[pl.ds(i + u * _LANES, _LANES)] = zero16

        @pl.loop(0, ep, step=_LANES)
        def _(i):
            s = src_v[pl.ds(i, _LANES)]
            d = dst_v[pl.ds(i, _LANES)]
            for c in range(w):
                si = (s + c * n_nodes) if c else s
                di = (d + c * n_nodes) if c else d
                v = plsc.load_gather(z_v, [si])
                plsc.addupdate_scatter(acc_v, [di], v)

        pltpu.sync_copy(acc_v, out_hbm.at[wid])

    return agg_kernel


def _tc1_body(deg_parts_ref, x_ref, w1t_ref, dinv_ref, z1_ref):
    deg = jnp.sum(deg_parts_ref[...], axis=0) + 1.0
    dinv = lax.rsqrt(deg)[None, :]
    dinv_ref[...] = dinv
    xwt = lax.dot_general(w1t_ref[...], x_ref[...],
                          (((1,), (1,)), ((), ())),
                          precision=_HIGH,
                          preferred_element_type=jnp.float32)
    z1_ref[...] = xwt * dinv


def _tc_layer_body(parts_ref, z_ref, dinv_ref, wt_ref, bcol_ref, zout_ref):
    acc = jnp.sum(parts_ref[...], axis=0)
    dinv = dinv_ref[...]
    h = jnp.maximum(dinv * (acc + z_ref[...]) + bcol_ref[...], 0.0)
    zout_ref[...] = lax.dot_general(wt_ref[...], h,
                                    (((1,), (0,)), ((), ())),
                                    precision=_HIGH,
                                    preferred_element_type=jnp.float32) * dinv


def _tc_final_body(parts_ref, z_ref, dinv_ref, wct_ref, b3col_ref, bccol_ref,
                   h_ref, out_ref):
    acc = jnp.sum(parts_ref[...], axis=0)
    dinv = dinv_ref[...]
    h = jnp.maximum(dinv * (acc + z_ref[...]) + b3col_ref[...], 0.0)
    h_ref[...] = h
    out_ref[...] = lax.dot_general(wct_ref[...], h,
                                   (((1,), (0,)), ((), ())),
                                   precision=_HIGH,
                                   preferred_element_type=jnp.float32) \
        + bccol_ref[...]


def _f32(shape):
    return jax.ShapeDtypeStruct(shape, jnp.float32)


def kernel(x, edge_index, W1, b1, W2, b2, W3, b3, Wc, bc):
    n, _ = x.shape
    e = edge_index.shape[1]
    w1 = W1.shape[1]
    w2 = W2.shape[1]
    w3 = W3.shape[1]
    c_out = Wc.shape[1]

    src = edge_index[0].astype(jnp.int32)
    dst = edge_index[1].astype(jnp.int32)

    deg_parts = _make_deg_kernel(n, e)(dst)

    dinv, z1 = pl.pallas_call(
        _tc1_body,
        out_shape=[_f32((1, n)), _f32((w1, n))],
    )(deg_parts, x, W1.T)

    p1 = _make_agg_kernel(n, e, w1)(z1.reshape(w1 * n), src, dst)
    z2 = pl.pallas_call(
        _tc_layer_body,
        out_shape=_f32((w2, n)),
    )(p1.reshape(NUM_WORKERS, w1, n), z1, dinv, W2.T, b1[:, None])

    p2 = _make_agg_kernel(n, e, w2)(z2.reshape(w2 * n), src, dst)
    z3 = pl.pallas_call(
        _tc_layer_body,
        out_shape=_f32((w3, n)),
    )(p2.reshape(NUM_WORKERS, w2, n), z2, dinv, W3.T, b2[:, None])

    p3 = _make_agg_kernel(n, e, w3)(z3.reshape(w3 * n), src, dst)
    ht, outt = pl.pallas_call(
        _tc_final_body,
        out_shape=[_f32((w3, n)), _f32((c_out, n))],
    )(p3.reshape(NUM_WORKERS, w3, n), z3, dinv, Wc.T, b3[:, None], bc[:, None])

    return (outt.T, ht.T)


# speedup vs baseline: 92.7605x; 1.3094x over previous
import dataclasses
import functools

import jax
import jax.numpy as jnp
from jax import lax
from jax.experimental import pallas as pl
from jax.experimental.pallas import tpu as pltpu
from jax.experimental.pallas import tpu_sc as plsc

NUM_WORKERS = 32
_LANES = 16

_cp = pltpu.CompilerParams()
if "needs_layout_passes" in pltpu.CompilerParams.__dataclass_fields__:
    _cp = dataclasses.replace(_cp, needs_layout_passes=False)

_MESH = plsc.VectorSubcoreMesh(core_axis_name="c", subcore_axis_name="s")

_PREC = jax.lax.Precision.HIGHEST


def _zero_fill(ref, total, unroll):
    zero16 = jnp.zeros((_LANES,), jnp.float32)
    step = unroll * _LANES
    assert total % step == 0

    @pl.loop(0, total, step=step)
    def _(i):
        for u in range(unroll):
            ref[pl.ds(i + u * _LANES, _LANES)] = zero16


@functools.lru_cache(maxsize=None)
def _make_deg_kernel(n_nodes: int, n_edges: int):
    ep = n_edges // NUM_WORKERS
    assert n_edges % NUM_WORKERS == 0 and ep % _LANES == 0
    ep_main = (ep // 32) * 32

    @functools.partial(
        pl.kernel,
        out_type=jax.ShapeDtypeStruct((NUM_WORKERS, n_nodes), jnp.float32),
        mesh=_MESH,
        scratch_types=[
            pltpu.VMEM((ep,), jnp.int32),
            pltpu.VMEM((n_nodes,), jnp.float32),
            pltpu.SemaphoreType.DMA,
        ],
        compiler_params=_cp,
    )
    def deg_kernel(dst_hbm, out_hbm, dst_v, deg_v, sem):
        wid = lax.axis_index("s") * 2 + lax.axis_index("c")
        cp_d = pltpu.async_copy(dst_hbm.at[pl.ds(wid * ep, ep)], dst_v, sem)
        _zero_fill(deg_v, n_nodes, 5)
        cp_d.wait()

        ones16 = jnp.ones((_LANES,), jnp.float32)

        @pl.loop(0, ep_main, step=2 * _LANES)
        def _(i):
            d0 = dst_v[pl.ds(i, _LANES)]
            d1 = dst_v[pl.ds(i + _LANES, _LANES)]
            plsc.addupdate_scatter(deg_v, [d0], ones16)
            plsc.addupdate_scatter(deg_v, [d1], ones16)

        for i in range(ep_main, ep, _LANES):
            plsc.addupdate_scatter(deg_v, [dst_v[pl.ds(i, _LANES)]], ones16)

        pltpu.sync_copy(deg_v, out_hbm.at[wid])

    return deg_kernel


@functools.lru_cache(maxsize=None)
def _make_agg_kernel(n_nodes: int, n_edges: int, w: int):
    ep = n_edges // NUM_WORKERS
    tbl = w * n_nodes
    assert n_edges % NUM_WORKERS == 0 and ep % _LANES == 0
    assert tbl % (10 * _LANES) == 0
    ep_main = (ep // 32) * 32

    @functools.partial(
        pl.kernel,
        out_type=jax.ShapeDtypeStruct((NUM_WORKERS, tbl), jnp.float32),
        mesh=_MESH,
        scratch_types=[
            pltpu.VMEM((ep,), jnp.int32),
            pltpu.VMEM((ep,), jnp.int32),
            pltpu.VMEM((tbl,), jnp.float32),
            pltpu.VMEM((tbl,), jnp.float32),
            pltpu.SemaphoreType.DMA,
            pltpu.SemaphoreType.DMA,
            pltpu.SemaphoreType.DMA,
        ],
        compiler_params=_cp,
    )
    def agg_kernel(z_hbm, src_hbm, dst_hbm, out_hbm,
                   src_v, dst_v, z_v, acc_v, sem_s, sem_d, sem_z):
        wid = lax.axis_index("s") * 2 + lax.axis_index("c")
        base = wid * ep
        cp_s = pltpu.async_copy(src_hbm.at[pl.ds(base, ep)], src_v, sem_s)
        cp_d = pltpu.async_copy(dst_hbm.at[pl.ds(base, ep)], dst_v, sem_d)
        cp_z = pltpu.async_copy(z_hbm, z_v, sem_z)
        _zero_fill(acc_v, tbl, 10)
        cp_s.wait()
        cp_d.wait()
        cp_z.wait()

        def do_group(s, d):
            vals = []
            for c in range(w):
                si = (s + c * n_nodes) if c else s
                vals.append((plsc.load_gather(z_v, [si]),
                             (d + c * n_nodes) if c else d))
            for v, di in vals:
                plsc.addupdate_scatter(acc_v, [di], v)

        @pl.loop(0, ep_main, step=2 * _LANES)
        def _(i):
            s0 = src_v[pl.ds(i, _LANES)]
            d0 = dst_v[pl.ds(i, _LANES)]
            s1 = src_v[pl.ds(i + _LANES, _LANES)]
            d1 = dst_v[pl.ds(i + _LANES, _LANES)]
            do_group(s0, d0)
            do_group(s1, d1)

        for i in range(ep_main, ep, _LANES):
            do_group(src_v[pl.ds(i, _LANES)], dst_v[pl.ds(i, _LANES)])

        pltpu.sync_copy(acc_v, out_hbm.at[wid])

    return agg_kernel


def _rows2d(flat, w, n):
    return jnp.concatenate([flat[c * n:(c + 1) * n][None, :] for c in range(w)],
                           axis=0)


def _tc1_body(deg_parts_ref, x_ref, w1t_ref, dinv_ref, z1_ref):
    n = x_ref.shape[0]
    w1 = w1t_ref.shape[0]
    deg = jnp.sum(deg_parts_ref[...], axis=0) + 1.0
    dinv = lax.rsqrt(deg)
    dinv_ref[...] = dinv
    xwt = lax.dot_general(w1t_ref[...], x_ref[...],
                          (((1,), (1,)), ((), ())),
                          precision=_PREC,
                          preferred_element_type=jnp.float32)
    for c in range(w1):
        z1_ref[pl.ds(c * n, n)] = xwt[c] * dinv


def _tc_layer_body(parts_ref, z_ref, dinv_ref, wt_ref, bcol_ref, zout_ref):
    w_out, w = wt_ref.shape
    n = dinv_ref.shape[0]
    flat = jnp.sum(parts_ref[...], axis=0)
    acc = _rows2d(flat + z_ref[...], w, n)
    dinv = dinv_ref[...][None, :]
    h = jnp.maximum(dinv * acc + bcol_ref[...], 0.0)
    zout = lax.dot_general(wt_ref[...], h,
                           (((1,), (0,)), ((), ())),
                           precision=_PREC,
                           preferred_element_type=jnp.float32) * dinv
    for c in range(w_out):
        zout_ref[pl.ds(c * n, n)] = zout[c]


def _tc_final_body(parts_ref, z_ref, dinv_ref, wc_ref, b3col_ref, bcrow_ref,
                   h_ref, out_ref):
    w = b3col_ref.shape[0]
    n = dinv_ref.shape[0]
    flat = jnp.sum(parts_ref[...], axis=0)
    acc = _rows2d(flat + z_ref[...], w, n)
    dinv = dinv_ref[...][None, :]
    h = jnp.maximum(dinv * acc + b3col_ref[...], 0.0)
    h_ref[...] = lax.dot_general(h, jnp.eye(w, dtype=jnp.float32),
                                 (((0,), (0,)), ((), ())),
                                 precision=_PREC,
                                 preferred_element_type=jnp.float32)
    out_ref[...] = lax.dot_general(h, wc_ref[...],
                                   (((0,), (0,)), ((), ())),
                                   precision=_PREC,
                                   preferred_element_type=jnp.float32) \
        + bcrow_ref[...]


def _f32(shape):
    return jax.ShapeDtypeStruct(shape, jnp.float32)


def kernel(x, edge_index, W1, b1, W2, b2, W3, b3, Wc, bc):
    n, _ = x.shape
    e = edge_index.shape[1]
    w1 = W1.shape[1]
    w2 = W2.shape[1]
    w3 = W3.shape[1]
    c_out = Wc.shape[1]

    src = edge_index[0].astype(jnp.int32)
    dst = edge_index[1].astype(jnp.int32)

    deg_parts = _make_deg_kernel(n, e)(dst)

    dinv, z1 = pl.pallas_call(
        _tc1_body,
        out_shape=[_f32((n,)), _f32((w1 * n,))],
    )(deg_parts, x, W1.T)

    p1 = _make_agg_kernel(n, e, w1)(z1, src, dst)
    z2 = pl.pallas_call(
        _tc_layer_body,
        out_shape=_f32((w2 * n,)),
    )(p1, z1, dinv, W2.T, b1[:, None])

    p2 = _make_agg_kernel(n, e, w2)(z2, src, dst)
    z3 = pl.pallas_call(
        _tc_layer_body,
        out_shape=_f32((w3 * n,)),
    )(p2, z2, dinv, W3.T, b2[:, None])

    p3 = _make_agg_kernel(n, e, w3)(z3, src, dst)
    h, out = pl.pallas_call(
        _tc_final_body,
        out_shape=[_f32((n, w3)), _f32((n, c_out))],
    )(p3, z3, dinv, Wc, b3[:, None], bc[None, :])

    return (out, h)
